# baseline (speedup 1.0000x reference)
import jax
import jax.numpy as jnp
from jax import lax
from jax.experimental import pallas as pl
from jax.experimental.pallas import tpu as pltpu
from jax.experimental.pallas import tpu_sc as plsc

N_NODES = 100000
N_EDGES = 6400000
W = 8
BLK = 128
NBLK = N_EDGES // BLK
NW = 32
CHUNK_BLKS = 16
CHUNK_E = CHUNK_BLKS * BLK
FULL_CHUNKS = (NBLK // NW) // CHUNK_BLKS
MAIN_BLKS = FULL_CHUNKS * CHUNK_BLKS
UNITS = NBLK // 8
N_ACC = 100352
RPT = N_ACC // 16
ZB_ROWS = 784
ZB_COPIES = RPT // ZB_ROWS


def _sc_body(fst_hbm, ev_hbm, idx_hbm, out_hbm,
             idx_v, fst_v, ev_v, val_v, zb_v, acc):
    cid = lax.axis_index("c")
    sid = lax.axis_index("s")
    wid = sid * 2 + cid

    lanes = lax.iota(jnp.int32, 16)
    lanes3 = lanes * 3
    zero16 = jnp.zeros((16,), jnp.float32)

    for c in range(W):
        colc = jnp.full((16,), c, jnp.int32)

        @pl.loop(0, ZB_ROWS // 16)
        def _(i):
            plsc.store_scatter(zb_v, [lanes + i * 16, colc], zero16)

    @pl.loop(0, ZB_COPIES)
    def _(q):
        pltpu.sync_copy(zb_v, acc.at[pl.ds(sid * RPT + q * ZB_ROWS, ZB_ROWS)])
    plsc.subcore_barrier()

    def process_block(k):
        base128 = k * BLK
        for i in range(8):
            base = base128 + i * 16
            f = fst_v[pl.ds(base, 16)]
            rows = lanes + base
            for c in range(3):
                ev_c = plsc.load_gather(ev_v, [lanes3 + (3 * base + c)])
                plsc.store_scatter(
                    val_v, [rows, jnp.full((16,), c, jnp.int32)], f * ev_c)
        pltpu.sync_copy(val_v.at[pl.ds(base128, BLK)],
                        acc.at[idx_v.at[k]], add=True)

    for c in range(3, W):
        colc = jnp.full((16,), c, jnp.int32)

        @pl.loop(0, CHUNK_E // 16)
        def _(i):
            plsc.store_scatter(val_v, [lanes + i * 16, colc], zero16)

    b0 = 8 * ((wid * UNITS) // NW)
    b1 = 8 * (((wid + 1) * UNITS) // NW)

    def stage_idx(e0):
        @pl.loop(0, CHUNK_BLKS)
        def _(k):
            pltpu.sync_copy(idx_hbm.at[pl.ds(e0 + k * BLK, BLK)],
                            idx_v.at[k])

    @pl.loop(0, FULL_CHUNKS)
    def _(ct):
        blk = b0 + ct * CHUNK_BLKS
        e0 = blk * BLK
        stage_idx(e0)
        pltpu.sync_copy(fst_hbm.at[pl.ds(e0, CHUNK_E)], fst_v)
        pltpu.sync_copy(ev_hbm.at[pl.ds(e0 * 3, CHUNK_E * 3)], ev_v)

        @pl.loop(0, CHUNK_BLKS)
        def _(k):
            process_block(k)

    rs = b1 - CHUNK_BLKS
    skip = CHUNK_BLKS - (b1 - b0 - MAIN_BLKS)
    stage_idx(rs * BLK)
    pltpu.sync_copy(fst_hbm.at[pl.ds(rs * BLK, CHUNK_E)], fst_v)
    pltpu.sync_copy(ev_hbm.at[pl.ds(rs * BLK * 3, CHUNK_E * 3)], ev_v)

    @pl.loop(skip, CHUNK_BLKS)
    def _(k):
        process_block(k)

    plsc.subcore_barrier()
    pltpu.sync_copy(acc.at[pl.ds(sid * RPT, RPT)],
                    out_hbm.at[pl.ds(cid * N_ACC + sid * RPT, RPT)])


@jax.jit
def _sc_segment_sum(fst, ev, idx):
    mesh = plsc.VectorSubcoreMesh(
        core_axis_name="c", subcore_axis_name="s", num_cores=2,
        num_subcores=16)
    return pl.kernel(
        _sc_body,
        out_type=jax.ShapeDtypeStruct((2 * N_ACC, W), jnp.float32),
        mesh=mesh,
        compiler_params=pltpu.CompilerParams(
            needs_layout_passes=False, use_tc_tiling_on_sc=False),
        scratch_types=[
            pltpu.VMEM((CHUNK_BLKS, BLK), jnp.int32),
            pltpu.VMEM((CHUNK_E,), jnp.float32),
            pltpu.VMEM((CHUNK_E * 3,), jnp.float32),
            pltpu.VMEM((CHUNK_E, W), jnp.float32),
            pltpu.VMEM((ZB_ROWS, W), jnp.float32),
            pltpu.VMEM_SHARED((N_ACC, W), jnp.float32),
        ],
    )(fst, ev, idx)


def kernel(F_st, edge_vec, edge_idx, atomic_numbers):
    del atomic_numbers
    fst = jnp.reshape(F_st, (N_EDGES,))
    ev = jnp.reshape(edge_vec, (N_EDGES * 3,))
    idx = edge_idx.astype(jnp.int32)
    partials = _sc_segment_sum(fst, ev, idx)
    partials = jnp.reshape(partials, (2, N_ACC, W))
    return partials[0, :N_NODES, :3] + partials[1, :N_NODES, :3]


# speedup vs baseline: 4.4002x; 4.4002x over previous
import jax
import jax.numpy as jnp
from jax import lax
from jax.experimental import pallas as pl
from jax.experimental.pallas import tpu as pltpu
from jax.experimental.pallas import tpu_sc as plsc

N_NODES = 100000
N_EDGES = 6400000
W = 8
BLK = 128
NBLK = N_EDGES // BLK
NW = 32
CHUNK_BLKS = 16
CHUNK_E = CHUNK_BLKS * BLK
FULL_CHUNKS = (NBLK // NW) // CHUNK_BLKS
MAIN_BLKS = FULL_CHUNKS * CHUNK_BLKS
UNITS = NBLK // 8
N_ACC = 100352
RPT = N_ACC // 16
ZB_ROWS = 784
ZB_COPIES = RPT // ZB_ROWS


def _sc_body(fst_hbm, ev_hbm, idx_hbm, out_hbm,
             idx_v, fst_v, ev_v, val_v, zb_v, acc):
    cid = lax.axis_index("c")
    sid = lax.axis_index("s")
    wid = sid * 2 + cid

    lanes = lax.iota(jnp.int32, 16)
    lanes3 = lanes * 3
    zero16 = jnp.zeros((16,), jnp.float32)

    for c in range(W):
        colc = jnp.full((16,), c, jnp.int32)

        @pl.loop(0, ZB_ROWS // 16)
        def _(i):
            plsc.store_scatter(zb_v, [lanes + i * 16, colc], zero16)

    @pl.loop(0, ZB_COPIES)
    def _(q):
        pltpu.sync_copy(zb_v, acc.at[pl.ds(sid * RPT + q * ZB_ROWS, ZB_ROWS)])
    plsc.subcore_barrier()

    def process_block(k):
        base128 = k * BLK
        for i in range(8):
            base = base128 + i * 16
            f = fst_v[pl.ds(base, 16)]
            rows = lanes + base
            for c in range(3):
                ev_c = ev_v[pl.ds(c * CHUNK_E + base, 16)]
                plsc.store_scatter(
                    val_v, [rows, jnp.full((16,), c, jnp.int32)], f * ev_c)
        pltpu.sync_copy(val_v.at[pl.ds(base128, BLK)],
                        acc.at[idx_v.at[k]], add=True)

    for c in range(3, W):
        colc = jnp.full((16,), c, jnp.int32)

        @pl.loop(0, CHUNK_E // 16)
        def _(i):
            plsc.store_scatter(val_v, [lanes + i * 16, colc], zero16)

    b0 = 8 * ((wid * UNITS) // NW)
    b1 = 8 * (((wid + 1) * UNITS) // NW)

    def stage(blk, e0):
        pltpu.sync_copy(idx_hbm.at[pl.ds(blk, CHUNK_BLKS)], idx_v)
        pltpu.sync_copy(fst_hbm.at[pl.ds(e0, CHUNK_E)], fst_v)
        for c in range(3):
            pltpu.sync_copy(ev_hbm.at[pl.ds(c * N_EDGES + e0, CHUNK_E)],
                            ev_v.at[pl.ds(c * CHUNK_E, CHUNK_E)])

    @pl.loop(0, FULL_CHUNKS)
    def _(ct):
        blk = b0 + ct * CHUNK_BLKS
        stage(blk, blk * BLK)

        @pl.loop(0, CHUNK_BLKS)
        def _(k):
            process_block(k)

    rs = b1 - CHUNK_BLKS
    skip = CHUNK_BLKS - (b1 - b0 - MAIN_BLKS)
    stage(rs, rs * BLK)

    @pl.loop(skip, CHUNK_BLKS)
    def _(k):
        process_block(k)

    plsc.subcore_barrier()
    pltpu.sync_copy(acc.at[pl.ds(sid * RPT, RPT)],
                    out_hbm.at[pl.ds(cid * N_ACC + sid * RPT, RPT)])


@jax.jit
def _sc_segment_sum(fst, ev, idx2d):
    mesh = plsc.VectorSubcoreMesh(
        core_axis_name="c", subcore_axis_name="s", num_cores=2,
        num_subcores=16)
    return pl.kernel(
        _sc_body,
        out_type=jax.ShapeDtypeStruct((2 * N_ACC, W), jnp.float32),
        mesh=mesh,
        compiler_params=pltpu.CompilerParams(
            needs_layout_passes=False, use_tc_tiling_on_sc=False),
        scratch_types=[
            pltpu.VMEM((CHUNK_BLKS, BLK), jnp.int32),
            pltpu.VMEM((CHUNK_E,), jnp.float32),
            pltpu.VMEM((CHUNK_E * 3,), jnp.float32),
            pltpu.VMEM((CHUNK_E, W), jnp.float32),
            pltpu.VMEM((ZB_ROWS, W), jnp.float32),
            pltpu.VMEM_SHARED((N_ACC, W), jnp.float32),
        ],
    )(fst, ev, idx2d)


def kernel(F_st, edge_vec, edge_idx, atomic_numbers):
    del atomic_numbers
    fst = jnp.reshape(F_st, (N_EDGES,))
    ev = jnp.reshape(jnp.transpose(edge_vec), (N_EDGES * 3,))
    idx2d = jnp.reshape(edge_idx.astype(jnp.int32), (NBLK, BLK))
    partials = _sc_segment_sum(fst, ev, idx2d)
    partials = jnp.reshape(partials, (2, N_ACC, W))
    return partials[0, :N_NODES, :3] + partials[1, :N_NODES, :3]


# speedup vs baseline: 4.4038x; 1.0008x over previous
import jax
import jax.numpy as jnp
from jax import lax
from jax.experimental import pallas as pl
from jax.experimental.pallas import tpu as pltpu
from jax.experimental.pallas import tpu_sc as plsc

N_NODES = 100000
N_EDGES = 6400000
W = 8
BLK = 128
NBLK = N_EDGES // BLK
NW = 32
CHUNK_BLKS = 16
CHUNK_E = CHUNK_BLKS * BLK
FULL_CHUNKS = (NBLK // NW) // CHUNK_BLKS
MAIN_BLKS = FULL_CHUNKS * CHUNK_BLKS
UNITS = NBLK // 8
N_ACC = 100352
RPT = N_ACC // 16
ZB_ROWS = 784
ZB_COPIES = RPT // ZB_ROWS


def _sc_body(fst_hbm, ev_hbm, idx_hbm, out_hbm,
             idx_v, fst_v, ev_v, val_v, zb_v, acc):
    cid = lax.axis_index("c")
    sid = lax.axis_index("s")
    wid = sid * 2 + cid

    lanes = lax.iota(jnp.int32, 16)
    lanes3 = lanes * 3
    zero16 = jnp.zeros((16,), jnp.float32)

    for c in range(W):
        colc = jnp.full((16,), c, jnp.int32)

        @pl.loop(0, ZB_ROWS // 16)
        def _(i):
            plsc.store_scatter(zb_v, [lanes + i * 16, colc], zero16)

    @pl.loop(0, ZB_COPIES)
    def _(q):
        pltpu.sync_copy(zb_v, acc.at[pl.ds(sid * RPT + q * ZB_ROWS, ZB_ROWS)])
    plsc.subcore_barrier()

    def process_block(k):
        base128 = k * BLK
        for i in range(8):
            base = base128 + i * 16
            f = fst_v[pl.ds(base, 16)]
            rows = lanes + base
            for c in range(3):
                ev_c = ev_v[pl.ds(c * CHUNK_E + base, 16)]
                plsc.store_scatter(
                    val_v, [rows, jnp.full((16,), c, jnp.int32)], f * ev_c)
        pltpu.sync_copy(val_v.at[pl.ds(base128, BLK)],
                        acc.at[idx_v.at[k]], add=True)

    for c in range(3, W):
        colc = jnp.full((16,), c, jnp.int32)

        @pl.loop(0, CHUNK_E // 16)
        def _(i):
            plsc.store_scatter(val_v, [lanes + i * 16, colc], zero16)

    b0 = 8 * ((wid * UNITS) // NW)
    b1 = 8 * (((wid + 1) * UNITS) // NW)

    def stage(blk, e0):
        pltpu.sync_copy(idx_hbm.at[pl.ds(blk, CHUNK_BLKS)], idx_v)
        pltpu.sync_copy(fst_hbm.at[pl.ds(e0, CHUNK_E)], fst_v)
        for c in range(3):
            pltpu.sync_copy(ev_hbm.at[pl.ds(c * N_EDGES + e0, CHUNK_E)],
                            ev_v.at[pl.ds(c * CHUNK_E, CHUNK_E)])

    @pl.loop(0, FULL_CHUNKS)
    def _(ct):
        blk = b0 + ct * CHUNK_BLKS
        stage(blk, blk * BLK)

        @pl.loop(0, CHUNK_BLKS)
        def _(k):
            process_block(k)

    rs = b1 - CHUNK_BLKS
    skip = CHUNK_BLKS - (b1 - b0 - MAIN_BLKS)
    stage(rs, rs * BLK)

    @pl.loop(skip, CHUNK_BLKS)
    def _(k):
        process_block(k)

    plsc.subcore_barrier()
    pltpu.sync_copy(acc.at[pl.ds(sid * RPT, RPT)],
                    out_hbm.at[pl.ds(cid * N_ACC + sid * RPT, RPT)])


@jax.jit
def _sc_segment_sum(fst, ev, idx2d):
    mesh = plsc.VectorSubcoreMesh(
        core_axis_name="c", subcore_axis_name="s", num_cores=2,
        num_subcores=16)
    return pl.kernel(
        _sc_body,
        out_type=jax.ShapeDtypeStruct((2 * N_ACC, W), jnp.float32),
        mesh=mesh,
        compiler_params=pltpu.CompilerParams(
            needs_layout_passes=False, use_tc_tiling_on_sc=False,
            skip_device_barrier=True),
        scratch_types=[
            pltpu.VMEM((CHUNK_BLKS, BLK), jnp.int32),
            pltpu.VMEM((CHUNK_E,), jnp.float32),
            pltpu.VMEM((CHUNK_E * 3,), jnp.float32),
            pltpu.VMEM((CHUNK_E, W), jnp.float32),
            pltpu.VMEM((ZB_ROWS, W), jnp.float32),
            pltpu.VMEM_SHARED((N_ACC, W), jnp.float32),
        ],
    )(fst, ev, idx2d)


def kernel(F_st, edge_vec, edge_idx, atomic_numbers):
    del atomic_numbers
    fst = jnp.reshape(F_st, (N_EDGES,))
    ev = jnp.reshape(jnp.transpose(edge_vec), (N_EDGES * 3,))
    idx2d = jnp.reshape(edge_idx.astype(jnp.int32), (NBLK, BLK))
    partials = _sc_segment_sum(fst, ev, idx2d)
    partials = jnp.reshape(partials, (2, N_ACC, W))
    return partials[0, :N_NODES, :3] + partials[1, :N_NODES, :3]
